# Initial kernel scaffold; baseline (speedup 1.0000x reference)
import jax
import jax.numpy as jnp
from jax.experimental import pallas as pl


def kernel(x, edge_index, batch, batch_ptr, params):
    raise NotImplementedError("write your pallas kernel here")


# speedup vs baseline: 11.7154x; 11.7154x over previous
import functools

import jax
import jax.numpy as jnp
from jax import lax
from jax.experimental import pallas as pl
from jax.experimental.pallas import tpu as pltpu
from jax.experimental.pallas import tpu_sc as plsc

_N = 10000
_E = 320000
_D = 128
_K1 = 64
_K2 = 16

_NC = 2
_NS = 16
_NW = _NC * _NS
_C = 128
_EPC = _E // _NC
_EPS = _E // _NW
_NCH = _EPS // _C
_TAIL = _EPS - _NCH * _C
_RPS = 624
_RLAST = _N - _NS * _RPS

_F32 = jnp.float32


def _sc_mesh():
    return plsc.VectorSubcoreMesh(core_axis_name="c", subcore_axis_name="s")


@functools.cache
def _agg_call(width):
    @functools.partial(
        pl.kernel,
        mesh=_sc_mesh(),
        out_type=jax.ShapeDtypeStruct((_NC * _N, width), _F32),
        scratch_types=[
            pltpu.VMEM((_C,), jnp.int32),
            pltpu.VMEM((_C,), jnp.int32),
            pltpu.VMEM((_C, width), _F32),
            pltpu.VMEM((_TAIL,), jnp.int32),
            pltpu.VMEM((_TAIL,), jnp.int32),
            pltpu.VMEM((_TAIL, width), _F32),
            pltpu.VMEM_SHARED((_N, width), _F32),
            pltpu.SemaphoreType.DMA,
        ],
    )
    def agg(table, gidx, sidx, zeros, out,
            gv, sv, rows_v, gvt, svt, rowst_v, acc, sem):
        cid = lax.axis_index("c")
        sid = lax.axis_index("s")
        rbase = sid * _RPS

        def stripe(src_ref, dst_ref):
            pltpu.sync_copy(src_ref.at[pl.ds(rbase, _RPS)],
                            dst_ref.at[pl.ds(rbase, _RPS)])

            @pl.when(sid == _NS - 1)
            def _():
                pltpu.sync_copy(src_ref.at[pl.ds(_NS * _RPS, _RLAST)],
                                dst_ref.at[pl.ds(_NS * _RPS, _RLAST)])

        stripe(zeros, acc)
        plsc.subcore_barrier()

        ebase = cid * _EPC + sid * _EPS

        @pl.loop(0, _NCH)
        def _(ci):
            base = ebase + ci * _C
            pltpu.sync_copy(gidx.at[pl.ds(base, _C)], gv)
            pltpu.sync_copy(sidx.at[pl.ds(base, _C)], sv)
            pltpu.async_copy(table.at[gv], rows_v, sem).wait()
            pltpu.sync_copy(rows_v, acc.at[sv], add=True)

        tbase = ebase + _NCH * _C
        pltpu.sync_copy(gidx.at[pl.ds(tbase, _TAIL)], gvt)
        pltpu.sync_copy(sidx.at[pl.ds(tbase, _TAIL)], svt)
        pltpu.async_copy(table.at[gvt], rowst_v, sem).wait()
        pltpu.sync_copy(rowst_v, acc.at[svt], add=True)

        plsc.subcore_barrier()
        stripe(acc, out.at[pl.ds(cid * _N, _N)])

    return agg


@functools.cache
def _deg_call():
    @functools.partial(
        pl.kernel,
        mesh=_sc_mesh(),
        out_type=jax.ShapeDtypeStruct((_NC * _N, _D), _F32),
        scratch_types=[
            pltpu.VMEM((_C,), jnp.int32),
            pltpu.VMEM((_TAIL,), jnp.int32),
            pltpu.VMEM((_C, _D), _F32),
            pltpu.VMEM_SHARED((_N, _D), _F32),
            pltpu.SemaphoreType.DMA,
        ],
    )
    def deg(sidx, ones, zeros, out, sv, svt, ones_v, acc, sem):
        cid = lax.axis_index("c")
        sid = lax.axis_index("s")
        rbase = sid * _RPS

        def stripe(src_ref, dst_ref):
            pltpu.sync_copy(src_ref.at[pl.ds(rbase, _RPS)],
                            dst_ref.at[pl.ds(rbase, _RPS)])

            @pl.when(sid == _NS - 1)
            def _():
                pltpu.sync_copy(src_ref.at[pl.ds(_NS * _RPS, _RLAST)],
                                dst_ref.at[pl.ds(_NS * _RPS, _RLAST)])

        pltpu.sync_copy(ones, ones_v)
        stripe(zeros, acc)
        plsc.subcore_barrier()

        ebase = cid * _EPC + sid * _EPS

        @pl.loop(0, _NCH)
        def _(ci):
            base = ebase + ci * _C
            pltpu.sync_copy(sidx.at[pl.ds(base, _C)], sv)
            pltpu.sync_copy(ones_v, acc.at[sv], add=True)

        tbase = ebase + _NCH * _C
        pltpu.sync_copy(sidx.at[pl.ds(tbase, _TAIL)], svt)
        pltpu.sync_copy(ones_v.at[pl.ds(0, _TAIL)], acc.at[svt], add=True)

        plsc.subcore_barrier()
        stripe(acc, out.at[pl.ds(cid * _N, _N)])

    return deg


def _tc(body, out_shapes):
    return pl.pallas_call(body, out_shape=out_shapes)


def _mm_body(x_ref, w_ref, o_ref):
    o_ref[...] = jnp.dot(x_ref[...], w_ref[...], preferred_element_type=_F32)


def _dinv_body(degs_ref, h_ref, dinv_ref, t_ref):
    deg = degs_ref[: _N, 0:1] + degs_ref[_N:, 0:1] + 1.0
    dinv = lax.rsqrt(deg)
    dinv_ref[...] = dinv
    t_ref[...] = h_ref[...] * dinv


def _bn_silu(y, g, b):
    mu = jnp.mean(y, axis=0, keepdims=True)
    var = jnp.mean((y - mu) ** 2, axis=0, keepdims=True)
    z = (y - mu) * lax.rsqrt(var + 1e-5) * g + b
    return z * jax.nn.sigmoid(z)


def _block1_body(acc_ref, t_ref, dinv_ref, b_ref, g_ref, be_ref, w_ref,
                 h1_ref, t1_ref):
    dinv = dinv_ref[...]
    gcn = dinv * (acc_ref[: _N] + acc_ref[_N:] + t_ref[...]) + b_ref[...]
    h1 = _bn_silu(gcn, g_ref[...], be_ref[...])
    h1_ref[...] = h1
    t1_ref[...] = jnp.dot(h1, w_ref[...], preferred_element_type=_F32) * dinv


def _block2_body(acc_ref, t_ref, dinv_ref, h1_ref, b_ref, g_ref, be_ref,
                 wp_ref, h2_ref, tp_ref):
    dinv = dinv_ref[...]
    gcn = dinv * (acc_ref[: _N] + acc_ref[_N:] + t_ref[...]) + b_ref[...]
    h2 = h1_ref[...] + _bn_silu(gcn, g_ref[...], be_ref[...])
    h2_ref[...] = h2
    tp_ref[...] = jnp.dot(h2, wp_ref[...], preferred_element_type=_F32) * dinv


def _softmax(rows):
    m = jnp.max(rows, axis=-1, keepdims=True)
    e = jnp.exp(rows - m)
    return e / jnp.sum(e, axis=-1, keepdims=True)


def _pool_body(acc_ref, tp_ref, dinv_ref, bp_ref, h2_ref, s_ref, xn_ref):
    dinv = dinv_ref[...]
    logits = (dinv * (acc_ref[: _N, : _K1] + acc_ref[_N:, : _K1]
                      + tp_ref[:, : _K1]) + bp_ref[...])
    s = _softmax(logits)
    s_ref[...] = jnp.concatenate([s, jnp.zeros((_N, _D - _K1), _F32)], axis=1)
    xn_ref[...] = lax.dot_general(s, h2_ref[...], (((0,), (0,)), ((), ())),
                                  preferred_element_type=_F32)


def _head_body(acc_ref, s_ref, xn_ref,
               w2_ref, b2_ref, g2_ref, be2_ref,
               w3_ref, b3_ref, g3_ref, be3_ref,
               wp2_ref, bp2_ref, wl_ref, bl_ref, logp_ref):
    a_s = acc_ref[: _N, : _K1] + acc_ref[_N:, : _K1]
    a_new = lax.dot_general(s_ref[:, : _K1], a_s, (((0,), (0,)), ((), ())),
                            preferred_element_type=_F32)
    ones = jnp.ones((_K1, 1), _F32)
    colsum = lax.dot_general(a_new, ones, (((0,), (0,)), ((), ())),
                             preferred_element_type=_F32)
    dinv2 = lax.rsqrt(colsum + 1.0)

    def conv(m, bias):
        agg = lax.dot_general(a_new, dinv2 * m, (((0,), (0,)), ((), ())),
                              preferred_element_type=_F32)
        return dinv2 * (agg + dinv2 * m) + bias

    h = xn_ref[...]
    y = conv(jnp.dot(h, w2_ref[...], preferred_element_type=_F32), b2_ref[...])
    h = _bn_silu(y, g2_ref[...], be2_ref[...])
    y = conv(jnp.dot(h, w3_ref[...], preferred_element_type=_F32), b3_ref[...])
    h = h + _bn_silu(y, g3_ref[...], be3_ref[...])
    lg = conv(jnp.dot(h, wp2_ref[...], preferred_element_type=_F32),
              bp2_ref[...])
    s2 = _softmax(lg)
    x2 = lax.dot_general(s2, h, (((0,), (0,)), ((), ())),
                         preferred_element_type=_F32)
    pooled = jnp.mean(x2, axis=0, keepdims=True)
    z = jnp.dot(pooled, wl_ref[...], preferred_element_type=_F32) + bl_ref[...]
    m = jnp.max(z, axis=-1, keepdims=True)
    lse = m + jnp.log(jnp.sum(jnp.exp(z - m), axis=-1, keepdims=True))
    logp_ref[...] = z - lse


def kernel(x, edge_index, batch, batch_ptr, params):
    p = params
    row = lambda v: v.reshape(1, -1)
    sd = lambda *s: jax.ShapeDtypeStruct(s, _F32)

    zeros_d = jnp.zeros((_N, _D), _F32)
    ones_cd = jnp.ones((_C, _D), _F32)
    wp1_pad = jnp.pad(p['Wp1'], ((0, 0), (0, _D - _K1)))

    src = edge_index[0]
    dst = edge_index[1]

    degs = _deg_call()(dst, ones_cd, zeros_d)
    h0m = _tc(_mm_body, sd(_N, _D))(x, p['W0'])
    dinv, t0 = _tc(_dinv_body, [sd(_N, 1), sd(_N, _D)])(degs, h0m)

    acc0 = _agg_call(_D)(t0, src, dst, zeros_d)
    h1, t1 = _tc(_block1_body, [sd(_N, _D), sd(_N, _D)])(
        acc0, t0, dinv, row(p['b0']), row(p['g0']), row(p['be0']), p['W1'])

    acc1 = _agg_call(_D)(t1, src, dst, zeros_d)
    h2, tp = _tc(_block2_body, [sd(_N, _D), sd(_N, _D)])(
        acc1, t1, dinv, h1, row(p['b1']), row(p['g1']), row(p['be1']), wp1_pad)

    accp = _agg_call(_D)(tp, src, dst, zeros_d)
    s, xn = _tc(_pool_body, [sd(_N, _D), sd(_K1, _D)])(
        accp, tp, dinv, row(p['bp1']), h2)

    acc_as = _agg_call(_D)(s, dst, src, zeros_d)
    logp = _tc(_head_body, sd(1, 10))(
        acc_as, s, xn,
        p['W2'], row(p['b2']), row(p['g2']), row(p['be2']),
        p['W3'], row(p['b3']), row(p['g3']), row(p['be3']),
        p['Wp2'], row(p['bp2']), p['Wl'], row(p['bl']))

    return logp, jnp.zeros((), _F32)
